# baseline (speedup 1.0000x reference)
import jax
import jax.numpy as jnp
from jax import lax
from jax.experimental import pallas as pl
from jax.experimental.pallas import tpu as pltpu
from jax.experimental.pallas import tpu_sc as plsc

D_MODEL = 64
SCALE = 8.0

NC = 2
NS = 16
NW = NC * NS
C = 56
K = 4
SPAD = 56
TW = 32768


def _prep_table(table):
    v, d = table.shape
    tab_t = table.T

    def body(t_ref, o_ref):
        o_ref[:, :D_MODEL] = t_ref[...].T * SCALE

    return pl.pallas_call(
        body,
        grid=((v + TW - 1) // TW,),
        in_specs=[pl.BlockSpec((d, TW), lambda j: (0, j))],
        out_specs=pl.BlockSpec((TW, 2 * D_MODEL), lambda j: (j, 0)),
        out_shape=jax.ShapeDtypeStruct((v, 2 * D_MODEL), jnp.float32),
        compiler_params=pltpu.CompilerParams(dimension_semantics=("parallel",)),
    )(tab_t)


def kernel(x, table):
    b, s = x.shape
    t128 = _prep_table(table)
    xp = jnp.concatenate([x, x[:, s - 6 :]], axis=1)
    n = b * SPAD
    idx = xp.reshape(n)
    bt = n // NW
    nch = bt // C
    g_total = nch // K
    assert n % (NW * C * K) == 0 and g_total % 2 == 0

    mesh = plsc.VectorSubcoreMesh(core_axis_name="core", subcore_axis_name="subcore")

    @pl.kernel(
        out_type=jax.ShapeDtypeStruct((n, 2 * D_MODEL), jnp.float32),
        mesh=mesh,
        compiler_params=pltpu.CompilerParams(use_tc_tiling_on_sc=False),
        scratch_types=(
            [pltpu.VMEM((bt,), jnp.int32)]
            + [pltpu.VMEM((C, 2 * D_MODEL), jnp.float32) for _ in range(2 * K)]
            + [pltpu.SemaphoreType.DMA for _ in range(4 * K + 1)]
        ),
    )
    def emb_kernel(tab_hbm, i_hbm, o_hbm, idx_v, *rest):
        bufs = rest[: 2 * K]
        gsems = rest[2 * K : 4 * K]
        osems = rest[4 * K : 6 * K]
        isem = rest[6 * K]
        wid = lax.axis_index("subcore") * NC + lax.axis_index("core")
        base = wid * bt
        pltpu.async_copy(i_hbm.at[pl.ds(base, bt)], idx_v, isem).wait()

        def gcopy(st, g, bb):
            c = g * K + bb
            return pltpu.make_async_copy(
                tab_hbm.at[idx_v.at[pl.ds(c * C, C)]], bufs[st + bb], gsems[st + bb]
            )

        def ocopy(st, g, bb):
            c = g * K + bb
            return pltpu.make_async_copy(
                bufs[st + bb].at[:, pl.ds(0, D_MODEL)],
                o_hbm.at[pl.ds(base + c * C, C), pl.ds(0, D_MODEL)],
                osems[st + bb],
            )

        def process(st, g, bb):
            gcopy(st, g, bb).wait()
            ocopy(st, g, bb).start()

        for bb in range(K):
            gcopy(0, 0, bb).start()

        @pl.loop(0, g_total, step=2)
        def _(g):
            process(0, g, 0)
            process(0, g, 1)

            @pl.when(g > 0)
            def _():
                for bb in range(K):
                    ocopy(K, g - 1, bb).wait()

            for bb in range(K):
                gcopy(K, g + 1, bb).start()
            process(0, g, 2)
            process(0, g, 3)

            process(K, g + 1, 0)
            process(K, g + 1, 1)

            @pl.when(g + 2 < g_total)
            def _():
                for bb in range(K):
                    ocopy(0, g, bb).wait()
                for bb in range(K):
                    gcopy(0, g + 2, bb).start()

            process(K, g + 1, 2)
            process(K, g + 1, 3)

        for bb in range(K):
            ocopy(0, g_total - 2, bb).wait()
        for bb in range(K):
            ocopy(K, g_total - 1, bb).wait()

    out = emb_kernel(t128, idx)
    return out.reshape(b, SPAD, 2 * D_MODEL)[:, :s, :D_MODEL]


# speedup vs baseline: 1.0092x; 1.0092x over previous
import jax
import jax.numpy as jnp
from jax import lax
from jax.experimental import pallas as pl
from jax.experimental.pallas import tpu as pltpu
from jax.experimental.pallas import tpu_sc as plsc

D_MODEL = 64
SCALE = 8.0

NC = 2
NS = 16
NW = NC * NS
C = 64
K = 4
SPAD = 56
TW = 32768


def _prep_table(table):
    v, d = table.shape
    tab_t = table.T

    def body(t_ref, o_ref):
        o_ref[:, :D_MODEL] = t_ref[...].T * SCALE

    return pl.pallas_call(
        body,
        grid=((v + TW - 1) // TW,),
        in_specs=[pl.BlockSpec((d, TW), lambda j: (0, j))],
        out_specs=pl.BlockSpec((TW, 2 * D_MODEL), lambda j: (j, 0)),
        out_shape=jax.ShapeDtypeStruct((v, 2 * D_MODEL), jnp.float32),
        compiler_params=pltpu.CompilerParams(dimension_semantics=("parallel",)),
    )(tab_t)


def kernel(x, table):
    b, s = x.shape
    t128 = _prep_table(table)
    xp = jnp.concatenate([x, x[:, s - 6 :]], axis=1)
    n = b * SPAD
    idx = xp.reshape(n)
    bt = n // NW
    nch = bt // C
    g_total = nch // K
    assert n % (NW * C * K) == 0 and g_total % 2 == 0

    mesh = plsc.VectorSubcoreMesh(core_axis_name="core", subcore_axis_name="subcore")

    @pl.kernel(
        out_type=jax.ShapeDtypeStruct((n, 2 * D_MODEL), jnp.float32),
        mesh=mesh,
        compiler_params=pltpu.CompilerParams(use_tc_tiling_on_sc=False),
        scratch_types=(
            [pltpu.VMEM((bt,), jnp.int32)]
            + [pltpu.VMEM((C, 2 * D_MODEL), jnp.float32) for _ in range(2 * K)]
            + [pltpu.SemaphoreType.DMA for _ in range(4 * K + 1)]
        ),
    )
    def emb_kernel(tab_hbm, i_hbm, o_hbm, idx_v, *rest):
        bufs = rest[: 2 * K]
        gsems = rest[2 * K : 4 * K]
        osems = rest[4 * K : 6 * K]
        isem = rest[6 * K]
        wid = lax.axis_index("subcore") * NC + lax.axis_index("core")
        base = wid * bt
        pltpu.async_copy(i_hbm.at[pl.ds(base, bt)], idx_v, isem).wait()

        def gcopy(st, g, bb):
            c = g * K + bb
            return pltpu.make_async_copy(
                tab_hbm.at[idx_v.at[pl.ds(c * C, C)]], bufs[st + bb], gsems[st + bb]
            )

        def ocopy(st, g, bb):
            c = g * K + bb
            return pltpu.make_async_copy(
                bufs[st + bb].at[:, pl.ds(0, D_MODEL)],
                o_hbm.at[pl.ds(base + c * C, C), pl.ds(0, D_MODEL)],
                osems[st + bb],
            )

        def process(st, g, bb):
            gcopy(st, g, bb).wait()
            ocopy(st, g, bb).start()

        for bb in range(K):
            gcopy(0, 0, bb).start()

        @pl.loop(0, g_total, step=2)
        def _(g):
            process(0, g, 0)
            process(0, g, 1)

            @pl.when(g > 0)
            def _():
                for bb in range(K):
                    ocopy(K, g - 1, bb).wait()

            for bb in range(K):
                gcopy(K, g + 1, bb).start()
            process(0, g, 2)
            process(0, g, 3)

            process(K, g + 1, 0)
            process(K, g + 1, 1)

            @pl.when(g + 2 < g_total)
            def _():
                for bb in range(K):
                    ocopy(0, g, bb).wait()
                for bb in range(K):
                    gcopy(0, g + 2, bb).start()

            process(K, g + 1, 2)
            process(K, g + 1, 3)

        for bb in range(K):
            ocopy(0, g_total - 2, bb).wait()
        for bb in range(K):
            ocopy(K, g_total - 1, bb).wait()

    out = emb_kernel(t128, idx)
    return out.reshape(b, SPAD, 2 * D_MODEL)[:, :s, :D_MODEL]
